# Initial kernel scaffold; baseline (speedup 1.0000x reference)
import jax
import jax.numpy as jnp
from jax.experimental import pallas as pl


def kernel(loc_data, conf_data, prior_data):
    raise NotImplementedError("write your pallas kernel here")


# speedup vs baseline: 26.6330x; 26.6330x over previous
import jax
import jax.numpy as jnp
from jax import lax
from jax.experimental import pallas as pl
from jax.experimental.pallas import tpu as pltpu

N = 20000
ROWS = 160
COLS = 128
NPAD = ROWS * COLS
NUM_CLASSES = 2
TOP_K = 750
CONF_THRESH = 0.75
NMS_THRESH = 0.3
NMS_TOP_K = 5000
VAR0 = 0.1
VAR1 = 0.2
ONE_BITS = 0x3F800000


def _nms_kernel(lx, ly, lw, lh, pcx, pcy, pw, ph, sc,
                out_ref, cur_ref, x1r, y1r, x2r, y2r, ar):
    f32 = jnp.float32
    neg_inf = f32(-jnp.inf)

    cx = pcx[...] + lx[...] * VAR0 * pw[...]
    cy = pcy[...] + ly[...] * VAR0 * ph[...]
    w = pw[...] * jnp.exp(lw[...] * VAR1)
    h = ph[...] * jnp.exp(lh[...] * VAR1)
    x1 = cx - w / 2
    y1 = cy - h / 2
    x2 = x1 + w
    y2 = y1 + h
    x1r[...] = x1
    y1r[...] = y1
    x2r[...] = x2
    y2r[...] = y2
    ar[...] = (x2 - x1) * (y2 - y1)

    scores = sc[...]
    valid = scores > CONF_THRESH
    bits = jax.lax.bitcast_convert_type(scores, jnp.int32)
    ridx = lax.broadcasted_iota(jnp.int32, (ROWS, COLS), 0)
    cidx = lax.broadcasted_iota(jnp.int32, (ROWS, COLS), 1)
    idx = ridx * COLS + cidx

    def bs_body(_, carry):
        lo, hi = carry
        mid = lo + (hi - lo + 1) // 2
        cnt = jnp.sum(jnp.where(valid & (bits >= mid), 1, 0))
        take = cnt >= NMS_TOP_K
        return jnp.where(take, mid, lo), jnp.where(take, hi, mid - 1)

    thr, _ = lax.fori_loop(0, 32, bs_body, (jnp.int32(0), jnp.int32(ONE_BITS)))

    c_gt = jnp.sum(jnp.where(valid & (bits > thr), 1, 0))
    m_need = NMS_TOP_K - c_gt
    tie = valid & (bits == thr)

    def bs2_body(_, carry):
        lo, hi = carry
        mid = lo + (hi - lo + 1) // 2
        cnt = jnp.sum(jnp.where(tie & (idx >= mid), 1, 0))
        take = cnt >= m_need
        return jnp.where(take, mid, lo), jnp.where(take, hi, mid - 1)

    cstar, _ = lax.fori_loop(0, 16, bs2_body, (jnp.int32(0), jnp.int32(NPAD)))

    active = valid & ((bits > thr) | (tie & (idx >= cstar)))
    cur_ref[...] = jnp.where(active, scores, neg_inf)
    out_ref[...] = jnp.zeros((TOP_K, 5), f32)

    def cond(carry):
        t, alive = carry
        return (t < TOP_K) & alive

    def body(carry):
        t, _ = carry
        cur = cur_ref[...]
        mval = jnp.max(cur)
        has = mval > neg_inf

        @pl.when(has)
        def _():
            i = jnp.max(jnp.where(cur == mval, idx, -1))
            sel = idx == i
            x1v = x1r[...]
            y1v = y1r[...]
            x2v = x2r[...]
            y2v = y2r[...]
            av = ar[...]
            zero = f32(0.0)
            x1i = jnp.sum(jnp.where(sel, x1v, zero))
            y1i = jnp.sum(jnp.where(sel, y1v, zero))
            x2i = jnp.sum(jnp.where(sel, x2v, zero))
            y2i = jnp.sum(jnp.where(sel, y2v, zero))
            ai = jnp.sum(jnp.where(sel, av, zero))
            xx1 = jnp.maximum(x1v, x1i)
            yy1 = jnp.maximum(y1v, y1i)
            xx2 = jnp.minimum(x2v, x2i)
            yy2 = jnp.minimum(y2v, y2i)
            iw = jnp.maximum(xx2 - xx1, zero)
            ih = jnp.maximum(yy2 - yy1, zero)
            inter = iw * ih
            union = av - inter + ai
            iou = inter / union
            supp = (iou > NMS_THRESH) | sel
            cur_ref[...] = jnp.where(supp, neg_inf, cur)
            out_ref[pl.ds(t, 1), 0:1] = jnp.full((1, 1), mval, f32)
            out_ref[pl.ds(t, 1), 1:2] = jnp.full((1, 1), x1i, f32)
            out_ref[pl.ds(t, 1), 2:3] = jnp.full((1, 1), y1i, f32)
            out_ref[pl.ds(t, 1), 3:4] = jnp.full((1, 1), x2i, f32)
            out_ref[pl.ds(t, 1), 4:5] = jnp.full((1, 1), y2i, f32)

        return t + 1, has

    lax.while_loop(cond, body, (jnp.int32(0), jnp.bool_(True)))


def _pad_col(x):
    return jnp.pad(x, (0, NPAD - N)).reshape(ROWS, COLS)


def kernel(loc_data, conf_data, prior_data):
    loc = jnp.asarray(loc_data).reshape(N, 4)
    conf = jnp.asarray(conf_data)
    priors = jnp.asarray(prior_data)
    args = [
        _pad_col(loc[:, 0]), _pad_col(loc[:, 1]),
        _pad_col(loc[:, 2]), _pad_col(loc[:, 3]),
        _pad_col(priors[:, 0]), _pad_col(priors[:, 1]),
        _pad_col(priors[:, 2]), _pad_col(priors[:, 3]),
        _pad_col(conf[:, 1]),
    ]
    rows = pl.pallas_call(
        _nms_kernel,
        out_shape=jax.ShapeDtypeStruct((TOP_K, 5), jnp.float32),
        scratch_shapes=[pltpu.VMEM((ROWS, COLS), jnp.float32)] * 6,
    )(*args)
    out = jnp.zeros((1, NUM_CLASSES, TOP_K, 5), jnp.float32)
    return out.at[0, 1].set(rows)
